# baseline (speedup 1.0000x reference)
import functools

import jax
import jax.numpy as jnp
from jax import lax
from jax.experimental import pallas as pl
from jax.experimental.pallas import tpu as pltpu
from jax.experimental.pallas import tpu_sc as plsc

_V = 1000001
_B = 16384
_D = 64
_NC = 2
_NS = 16
_NW = _NC * _NS
_BPW = _B // _NW

_mesh = plsc.VectorSubcoreMesh(core_axis_name="c", subcore_axis_name="s")


@functools.partial(
    pl.kernel,
    mesh=_mesh,
    out_type=jax.ShapeDtypeStruct((_D, _B), jnp.float32),
    scratch_types=[
        pltpu.VMEM((_BPW,), jnp.int32),
        pltpu.VMEM((_D * _BPW,), jnp.int32),
        pltpu.VMEM((_D * _BPW,), jnp.float32),
        pltpu.SemaphoreType.DMA,
    ],
    compiler_params=pltpu.CompilerParams(use_tc_tiling_on_sc=False),
)
def _embed_gather(labels_hbm, tablet_hbm, outt_hbm, idx_v, widx_v, cols_v, sem):
    wid = lax.axis_index("s") * _NC + lax.axis_index("c")
    base = wid * _BPW
    pltpu.sync_copy(labels_hbm.at[pl.ds(base, _BPW)], idx_v)

    def build(g, carry):
        vec = idx_v[pl.ds(g * 16, 16)]
        for d in range(_D):
            widx_v[pl.ds(d * _BPW + g * 16, 16)] = vec + d * _V
        return carry

    lax.fori_loop(0, _BPW // 16, build, 0)

    pltpu.async_copy(tablet_hbm.at[widx_v], cols_v, sem).wait()

    for d in range(_D):
        pltpu.async_copy(
            cols_v.at[pl.ds(d * _BPW, _BPW)],
            outt_hbm.at[d, pl.ds(base, _BPW)],
            sem,
        )
    for d in range(_D):
        pltpu.make_async_copy(
            cols_v.at[pl.ds(0, _BPW)],
            outt_hbm.at[0, pl.ds(0, _BPW)],
            sem,
        ).wait()


def kernel(labels, table):
    out_t = _embed_gather(labels.astype(jnp.int32), table.T.reshape(-1))
    return out_t.T
